# initial kernel scaffold (unmeasured)
import jax
import jax.numpy as jnp
from jax import lax
from jax.experimental import pallas as pl
from jax.experimental.pallas import tpu as pltpu

N_DEV = 8


def kernel(x, w_mat):
    m, k_per = x.shape
    _, n = w_mat.shape
    ch = m // N_DEV

    def body(x_ref, w_ref, out_ref, comm_ref, send_sems, recv_sems):
        me = lax.axis_index("i")
        left = (me - 1) % N_DEV
        right = (me + 1) % N_DEV

        barrier_sem = pltpu.get_barrier_semaphore()
        for nbr in (left, right):
            pl.semaphore_signal(
                barrier_sem, inc=1,
                device_id=(nbr,), device_id_type=pl.DeviceIdType.MESH,
            )
        pl.semaphore_wait(barrier_sem, 2)

        out_ref[...] = jnp.dot(
            x_ref[...], w_ref[...], preferred_element_type=jnp.float32
        )

        def rows(c):
            return pl.ds(c * ch, ch)

        for s in range(N_DEV - 1):
            send_c = (me - s) % N_DEV
            rdma = pltpu.make_async_remote_copy(
                src_ref=out_ref.at[rows(send_c), :],
                dst_ref=comm_ref.at[s],
                send_sem=send_sems.at[s],
                recv_sem=recv_sems.at[s],
                device_id=(right,),
                device_id_type=pl.DeviceIdType.MESH,
            )
            rdma.start()
            rdma.wait()
            recv_c = (me - s - 1) % N_DEV
            out_ref[rows(recv_c), :] += comm_ref[s]

        for t in range(N_DEV - 1):
            send_c = (me + 1 - t) % N_DEV
            rdma = pltpu.make_async_remote_copy(
                src_ref=out_ref.at[rows(send_c), :],
                dst_ref=out_ref.at[rows(send_c), :],
                send_sem=send_sems.at[N_DEV - 1 + t],
                recv_sem=recv_sems.at[N_DEV - 1 + t],
                device_id=(right,),
                device_id_type=pl.DeviceIdType.MESH,
            )
            rdma.start()
            rdma.wait()

        y = out_ref[...]
        amax = jnp.max(jnp.abs(y))
        scale = amax / 448.0
        q = (y / scale).astype(jnp.float8_e4m3fn)
        out_ref[...] = q.astype(jnp.float32) * scale

    return pl.pallas_call(
        body,
        out_shape=jax.ShapeDtypeStruct((m, n), jnp.float32),
        in_specs=[
            pl.BlockSpec(memory_space=pltpu.VMEM),
            pl.BlockSpec(memory_space=pltpu.VMEM),
        ],
        out_specs=pl.BlockSpec(memory_space=pltpu.VMEM),
        scratch_shapes=[
            pltpu.VMEM((N_DEV - 1, ch, n), jnp.float32),
            pltpu.SemaphoreType.DMA((2 * (N_DEV - 1),)),
            pltpu.SemaphoreType.DMA((2 * (N_DEV - 1),)),
        ],
        compiler_params=pltpu.CompilerParams(collective_id=0),
    )(x, w_mat)


# baseline (device time: 717665 ns/iter reference)
import jax
import jax.numpy as jnp
from jax import lax
from jax.experimental import pallas as pl
from jax.experimental.pallas import tpu as pltpu

N_DEV = 8
N_SLOTS = 2


def kernel(x, w_mat):
    m, k_per = x.shape
    _, n = w_mat.shape
    ch = m // N_DEV

    def body(x_ref, w_ref, out_ref, comm_ref, send_sems, recv_sems,
             credit_sems):
        me = lax.axis_index("i")
        left = (me - 1) % N_DEV
        right = (me + 1) % N_DEV

        barrier_sem = pltpu.get_barrier_semaphore()
        for nbr in (left, right):
            pl.semaphore_signal(
                barrier_sem, inc=1,
                device_id=(nbr,), device_id_type=pl.DeviceIdType.MESH,
            )
        pl.semaphore_wait(barrier_sem, 2)

        def gemm_body(c, carry):
            out_ref[pl.ds(c * ch, ch), :] = jnp.dot(
                x_ref[pl.ds(c * ch, ch), :], w_ref[...],
                preferred_element_type=jnp.float32,
            )
            return carry

        lax.fori_loop(0, N_DEV, gemm_body, 0)

        def rows(c):
            return pl.ds(c * ch, ch)

        for s in range(N_DEV - 1):
            slot = s % N_SLOTS
            send_c = (me - s) % N_DEV
            rdma = pltpu.make_async_remote_copy(
                src_ref=out_ref.at[rows(send_c), :],
                dst_ref=comm_ref.at[slot],
                send_sem=send_sems.at[s],
                recv_sem=recv_sems.at[s],
                device_id=(right,),
                device_id_type=pl.DeviceIdType.MESH,
            )
            if s >= N_SLOTS:
                pl.semaphore_wait(credit_sems.at[slot], 1)
            rdma.start()
            rdma.wait()
            recv_c = (me - s - 1) % N_DEV
            out_ref[rows(recv_c), :] += comm_ref[slot]
            if s + N_SLOTS <= N_DEV - 2:
                pl.semaphore_signal(
                    credit_sems.at[slot], inc=1,
                    device_id=(left,), device_id_type=pl.DeviceIdType.MESH,
                )

        for t in range(N_DEV - 1):
            send_c = (me + 1 - t) % N_DEV
            rdma = pltpu.make_async_remote_copy(
                src_ref=out_ref.at[rows(send_c), :],
                dst_ref=out_ref.at[rows(send_c), :],
                send_sem=send_sems.at[N_DEV - 1 + t],
                recv_sem=recv_sems.at[N_DEV - 1 + t],
                device_id=(right,),
                device_id_type=pl.DeviceIdType.MESH,
            )
            rdma.start()
            rdma.wait()

        def amax_body(c, cur):
            blk = out_ref[pl.ds(c * ch, ch), :]
            return jnp.maximum(cur, jnp.max(jnp.abs(blk)))

        amax = lax.fori_loop(0, N_DEV, amax_body, jnp.float32(0.0))
        scale = amax / 448.0
        inv_scale = 448.0 / amax

        def quant_body(c, carry):
            blk = out_ref[pl.ds(c * ch, ch), :]
            q = (blk * inv_scale).astype(jnp.float8_e4m3fn)
            out_ref[pl.ds(c * ch, ch), :] = q.astype(jnp.float32) * scale
            return carry

        lax.fori_loop(0, N_DEV, quant_body, 0)

    return pl.pallas_call(
        body,
        out_shape=jax.ShapeDtypeStruct((m, n), jnp.float32),
        in_specs=[
            pl.BlockSpec(memory_space=pltpu.VMEM),
            pl.BlockSpec(memory_space=pltpu.VMEM),
        ],
        out_specs=pl.BlockSpec(memory_space=pltpu.VMEM),
        scratch_shapes=[
            pltpu.VMEM((N_SLOTS, ch, n), jnp.float32),
            pltpu.SemaphoreType.DMA((2 * (N_DEV - 1),)),
            pltpu.SemaphoreType.DMA((2 * (N_DEV - 1),)),
            pltpu.SemaphoreType.REGULAR((N_SLOTS,)),
        ],
        compiler_params=pltpu.CompilerParams(
            collective_id=0,
            vmem_limit_bytes=56 * 1024 * 1024,
        ),
    )(x, w_mat)


# device time: 408739 ns/iter; 1.7558x vs baseline; 1.7558x over previous
import jax
import jax.numpy as jnp
from jax import lax
from jax.experimental import pallas as pl
from jax.experimental.pallas import tpu as pltpu

N_DEV = 8
N_SLOTS = 2


def kernel(x, w_mat):
    m, k_per = x.shape
    _, n = w_mat.shape
    ch = m // N_DEV
    half = ch // 2

    def body(x_ref, w_ref, out_ref, comm_ref, send_sems, recv_sems,
             credit_sems):
        me = lax.axis_index("i")
        left = (me - 1) % N_DEV
        right = (me + 1) % N_DEV

        barrier_sem = pltpu.get_barrier_semaphore()
        for nbr in (left, right):
            pl.semaphore_signal(
                barrier_sem, inc=1,
                device_id=(nbr,), device_id_type=pl.DeviceIdType.MESH,
            )
        pl.semaphore_wait(barrier_sem, 2)

        def gemm_body(c, carry):
            out_ref[pl.ds(c * ch, ch), :] = jnp.dot(
                x_ref[pl.ds(c * ch, ch), :], w_ref[...],
                preferred_element_type=jnp.float32,
            )
            return carry

        lax.fori_loop(0, N_DEV, gemm_body, 0)

        def rows_h(c, d):
            return pl.ds(c * ch + d * half, half)

        def rs_send_chunk(d, s):
            return (me - s) % N_DEV if d == 0 else (me + s) % N_DEV

        def rs_recv_chunk(d, s):
            return (me - s - 1) % N_DEV if d == 0 else (me + s + 1) % N_DEV

        def ag_send_chunk(d, t):
            return (me + 1 - t) % N_DEV if d == 0 else (me - 1 + t) % N_DEV

        to_nbr = (lambda d: right if d == 0 else left)
        from_nbr = (lambda d: left if d == 0 else right)

        for s in range(N_DEV - 1):
            slot = s % N_SLOTS
            rdmas = []
            for d in (0, 1):
                rdma = pltpu.make_async_remote_copy(
                    src_ref=out_ref.at[rows_h(rs_send_chunk(d, s), d), :],
                    dst_ref=comm_ref.at[d, slot],
                    send_sem=send_sems.at[d, s],
                    recv_sem=recv_sems.at[d, s],
                    device_id=(to_nbr(d),),
                    device_id_type=pl.DeviceIdType.MESH,
                )
                if s >= N_SLOTS:
                    pl.semaphore_wait(credit_sems.at[d, slot], 1)
                rdma.start()
                rdmas.append(rdma)
            for d in (0, 1):
                rdmas[d].wait()
                out_ref[rows_h(rs_recv_chunk(d, s), d), :] += comm_ref[d, slot]
                if s + N_SLOTS <= N_DEV - 2:
                    pl.semaphore_signal(
                        credit_sems.at[d, slot], inc=1,
                        device_id=(from_nbr(d),),
                        device_id_type=pl.DeviceIdType.MESH,
                    )

        for t in range(N_DEV - 1):
            rdmas = []
            for d in (0, 1):
                rc = rows_h(ag_send_chunk(d, t), d)
                rdma = pltpu.make_async_remote_copy(
                    src_ref=out_ref.at[rc, :],
                    dst_ref=out_ref.at[rc, :],
                    send_sem=send_sems.at[d, N_DEV - 1 + t],
                    recv_sem=recv_sems.at[d, N_DEV - 1 + t],
                    device_id=(to_nbr(d),),
                    device_id_type=pl.DeviceIdType.MESH,
                )
                rdma.start()
                rdmas.append(rdma)
            for d in (0, 1):
                rdmas[d].wait()

        def amax_body(c, cur):
            blk = out_ref[pl.ds(c * ch, ch), :]
            return jnp.maximum(cur, jnp.max(jnp.abs(blk)))

        amax = lax.fori_loop(0, N_DEV, amax_body, jnp.float32(0.0))
        scale = amax / 448.0
        inv_scale = 448.0 / amax

        def quant_body(c, carry):
            blk = out_ref[pl.ds(c * ch, ch), :]
            q = (blk * inv_scale).astype(jnp.float8_e4m3fn)
            out_ref[pl.ds(c * ch, ch), :] = q.astype(jnp.float32) * scale
            return carry

        lax.fori_loop(0, N_DEV, quant_body, 0)

    return pl.pallas_call(
        body,
        out_shape=jax.ShapeDtypeStruct((m, n), jnp.float32),
        in_specs=[
            pl.BlockSpec(memory_space=pltpu.VMEM),
            pl.BlockSpec(memory_space=pltpu.VMEM),
        ],
        out_specs=pl.BlockSpec(memory_space=pltpu.VMEM),
        scratch_shapes=[
            pltpu.VMEM((2, N_SLOTS, half, n), jnp.float32),
            pltpu.SemaphoreType.DMA((2, 2 * (N_DEV - 1))),
            pltpu.SemaphoreType.DMA((2, 2 * (N_DEV - 1))),
            pltpu.SemaphoreType.REGULAR((2, N_SLOTS)),
        ],
        compiler_params=pltpu.CompilerParams(
            collective_id=0,
            vmem_limit_bytes=56 * 1024 * 1024,
        ),
    )(x, w_mat)


# device time: 291229 ns/iter; 2.4643x vs baseline; 1.4035x over previous
import jax
import jax.numpy as jnp
from jax import lax
from jax.experimental import pallas as pl
from jax.experimental.pallas import tpu as pltpu

N_DEV = 8
AMAX_HOPS = (4, 3)


def kernel(x, w_mat):
    m, k_per = x.shape
    _, n = w_mat.shape
    ch = m // N_DEV
    half = ch // 2

    def body(x_ref, w_ref, out_ref, comm_ref, q_ref,
             amax_sbuf, amax_rbuf,
             rs_send_sems, rs_recv_sems, credit_sems,
             ax_send_sems, ax_recv_sems,
             ag_send_sems, ag_recv_sems):
        me = lax.axis_index("i")
        left = (me - 1) % N_DEV
        right = (me + 1) % N_DEV

        barrier_sem = pltpu.get_barrier_semaphore()
        for nbr in (left, right):
            pl.semaphore_signal(
                barrier_sem, inc=1,
                device_id=(nbr,), device_id_type=pl.DeviceIdType.MESH,
            )
        pl.semaphore_wait(barrier_sem, 2)

        def rows_h(c, d):
            return pl.ds(c * ch + d * half, half)

        def rs_send_chunk(d, s):
            return (me - s) % N_DEV if d == 0 else (me + s) % N_DEV

        def rs_recv_chunk(d, s):
            return (me - s - 1) % N_DEV if d == 0 else (me + s + 1) % N_DEV

        def ag_send_chunk(d, t):
            return (me + 1 - t) % N_DEV if d == 0 else (me - 1 + t) % N_DEV

        to_nbr = (lambda d: right if d == 0 else left)
        from_nbr = (lambda d: left if d == 0 else right)

        def start_rs(s):
            rdmas = []
            for d in (0, 1):
                rdma = pltpu.make_async_remote_copy(
                    src_ref=out_ref.at[rows_h(rs_send_chunk(d, s), d), :],
                    dst_ref=comm_ref.at[d],
                    send_sem=rs_send_sems.at[d, s],
                    recv_sem=rs_recv_sems.at[d, s],
                    device_id=(to_nbr(d),),
                    device_id_type=pl.DeviceIdType.MESH,
                )
                if s >= 1:
                    pl.semaphore_wait(credit_sems.at[d], 1)
                rdma.start()
                rdmas.append(rdma)
            return rdmas

        out_ref[pl.ds(me * ch, ch), :] = jnp.dot(
            x_ref[pl.ds(me * ch, ch), :], w_ref[...],
            preferred_element_type=jnp.float32,
        )
        rdmas = start_rs(0)

        def gemm_body(j, carry):
            c = (me - j) % N_DEV
            out_ref[pl.ds(c * ch, ch), :] = jnp.dot(
                x_ref[pl.ds(c * ch, ch), :], w_ref[...],
                preferred_element_type=jnp.float32,
            )
            return carry

        lax.fori_loop(1, N_DEV, gemm_body, 0)

        for s in range(N_DEV - 1):
            for d in (0, 1):
                rdmas[d].wait()
                out_ref[rows_h(rs_recv_chunk(d, s), d), :] += comm_ref[d]
                if s <= N_DEV - 3:
                    pl.semaphore_signal(
                        credit_sems.at[d], inc=1,
                        device_id=(from_nbr(d),),
                        device_id_type=pl.DeviceIdType.MESH,
                    )
            if s + 1 <= N_DEV - 2:
                rdmas = start_rs(s + 1)

        own0 = rows_h((me + 1) % N_DEV, 0)
        own1 = rows_h((me - 1) % N_DEV, 1)
        local_amax = jnp.maximum(
            jnp.max(jnp.abs(out_ref[own0, :])),
            jnp.max(jnp.abs(out_ref[own1, :])),
        )
        r = [local_amax, local_amax]
        for h in range(max(AMAX_HOPS)):
            rdmas = []
            active = [d for d in (0, 1) if h < AMAX_HOPS[d]]
            for d in active:
                amax_sbuf[d, :, :] = jnp.full((1, 128), r[d], jnp.float32)
                rdma = pltpu.make_async_remote_copy(
                    src_ref=amax_sbuf.at[d],
                    dst_ref=amax_rbuf.at[d, h],
                    send_sem=ax_send_sems.at[d, h],
                    recv_sem=ax_recv_sems.at[d, h],
                    device_id=(to_nbr(d),),
                    device_id_type=pl.DeviceIdType.MESH,
                )
                rdma.start()
                rdmas.append(rdma)
            for d, rdma in zip(active, rdmas):
                rdma.wait()
                r[d] = jnp.maximum(r[d], amax_rbuf[d, h, 0, 0])
        amax = jnp.maximum(r[0], r[1])
        scale = amax / 448.0
        inv_scale = 448.0 / amax

        for rows in (own0, own1):
            q_ref[rows, :] = (out_ref[rows, :] * inv_scale).astype(
                jnp.float8_e4m3fn
            )

        def start_ag(t):
            rdmas = []
            for d in (0, 1):
                rc = rows_h(ag_send_chunk(d, t), d)
                rdma = pltpu.make_async_remote_copy(
                    src_ref=q_ref.at[rc, :],
                    dst_ref=q_ref.at[rc, :],
                    send_sem=ag_send_sems.at[d, t],
                    recv_sem=ag_recv_sems.at[d, t],
                    device_id=(to_nbr(d),),
                    device_id_type=pl.DeviceIdType.MESH,
                )
                rdma.start()
                rdmas.append(rdma)
            return rdmas

        rdmas = start_ag(0)
        for t in range(N_DEV - 1):
            for d in (0, 1):
                rdmas[d].wait()
            if t + 1 <= N_DEV - 2:
                rdmas = start_ag(t + 1)

        def deq_body(c, carry):
            q = q_ref[pl.ds(c * ch, ch), :]
            out_ref[pl.ds(c * ch, ch), :] = q.astype(jnp.float32) * scale
            return carry

        lax.fori_loop(0, N_DEV, deq_body, 0)

    return pl.pallas_call(
        body,
        out_shape=jax.ShapeDtypeStruct((m, n), jnp.float32),
        in_specs=[
            pl.BlockSpec(memory_space=pltpu.VMEM),
            pl.BlockSpec(memory_space=pltpu.VMEM),
        ],
        out_specs=pl.BlockSpec(memory_space=pltpu.VMEM),
        scratch_shapes=[
            pltpu.VMEM((2, half, n), jnp.float32),
            pltpu.VMEM((m, n), jnp.float8_e4m3fn),
            pltpu.VMEM((2, 1, 128), jnp.float32),
            pltpu.VMEM((2, max(AMAX_HOPS), 1, 128), jnp.float32),
            pltpu.SemaphoreType.DMA((2, N_DEV - 1)),
            pltpu.SemaphoreType.DMA((2, N_DEV - 1)),
            pltpu.SemaphoreType.REGULAR((2,)),
            pltpu.SemaphoreType.DMA((2, max(AMAX_HOPS))),
            pltpu.SemaphoreType.DMA((2, max(AMAX_HOPS))),
            pltpu.SemaphoreType.DMA((2, N_DEV - 1)),
            pltpu.SemaphoreType.DMA((2, N_DEV - 1)),
        ],
        compiler_params=pltpu.CompilerParams(
            collective_id=0,
            vmem_limit_bytes=58 * 1024 * 1024,
        ),
    )(x, w_mat)


# device time: 212773 ns/iter; 3.3729x vs baseline; 1.3687x over previous
import jax
import jax.numpy as jnp
from jax import lax
from jax.experimental import pallas as pl
from jax.experimental.pallas import tpu as pltpu

N_DEV = 8
AMAX_HOPS = (4, 3)


def kernel(x, w_mat):
    m, k_per = x.shape
    _, n = w_mat.shape
    ch = m // N_DEV
    half = ch // 2

    def body(x_ref, w_ref, out_ref, comm_ref, sbuf_ref, q_ref,
             amax_sbuf, amax_rbuf,
             rs_send_sems, rs_recv_sems, credit_sems,
             ax_send_sems, ax_recv_sems,
             ag_send_sems, ag_recv_sems):
        me = lax.axis_index("i")
        left = (me - 1) % N_DEV
        right = (me + 1) % N_DEV

        barrier_sem = pltpu.get_barrier_semaphore()
        for nbr in (left, right):
            pl.semaphore_signal(
                barrier_sem, inc=1,
                device_id=(nbr,), device_id_type=pl.DeviceIdType.MESH,
            )
        pl.semaphore_wait(barrier_sem, 2)

        def rows_h(c, d):
            return pl.ds(c * ch + d * half, half)

        def rs_send_chunk(d, s):
            return (me - s) % N_DEV if d == 0 else (me + s) % N_DEV

        def rs_recv_chunk(d, s):
            return (me - s - 1) % N_DEV if d == 0 else (me + s + 1) % N_DEV

        def ag_send_chunk(d, t):
            return (me + 1 - t) % N_DEV if d == 0 else (me - 1 + t) % N_DEV

        to_nbr = (lambda d: right if d == 0 else left)
        from_nbr = (lambda d: left if d == 0 else right)

        def fill_sbuf(s):
            for d in (0, 1):
                sbuf_ref[d] = out_ref[
                    rows_h(rs_send_chunk(d, s), d), :
                ].astype(jnp.bfloat16)

        def start_rs(s):
            rdmas = []
            for d in (0, 1):
                rdma = pltpu.make_async_remote_copy(
                    src_ref=sbuf_ref.at[d],
                    dst_ref=comm_ref.at[d],
                    send_sem=rs_send_sems.at[d, s],
                    recv_sem=rs_recv_sems.at[d, s],
                    device_id=(to_nbr(d),),
                    device_id_type=pl.DeviceIdType.MESH,
                )
                if s >= 1:
                    pl.semaphore_wait(credit_sems.at[d], 1)
                rdma.start()
                rdmas.append(rdma)
            return rdmas

        out_ref[pl.ds(me * ch, ch), :] = jnp.dot(
            x_ref[pl.ds(me * ch, ch), :], w_ref[...],
            preferred_element_type=jnp.float32,
        )
        fill_sbuf(0)
        rdmas = start_rs(0)

        def gemm_body(j, carry):
            c = (me - j) % N_DEV
            out_ref[pl.ds(c * ch, ch), :] = jnp.dot(
                x_ref[pl.ds(c * ch, ch), :], w_ref[...],
                preferred_element_type=jnp.float32,
            )
            return carry

        lax.fori_loop(1, N_DEV, gemm_body, 0)

        for s in range(N_DEV - 1):
            for d in (0, 1):
                rdmas[d].wait()
                out_ref[rows_h(rs_recv_chunk(d, s), d), :] += (
                    comm_ref[d].astype(jnp.float32)
                )
                if s <= N_DEV - 3:
                    pl.semaphore_signal(
                        credit_sems.at[d], inc=1,
                        device_id=(from_nbr(d),),
                        device_id_type=pl.DeviceIdType.MESH,
                    )
            if s + 1 <= N_DEV - 2:
                fill_sbuf(s + 1)
                rdmas = start_rs(s + 1)

        own0 = rows_h((me + 1) % N_DEV, 0)
        own1 = rows_h((me - 1) % N_DEV, 1)
        local_amax = jnp.maximum(
            jnp.max(jnp.abs(out_ref[own0, :])),
            jnp.max(jnp.abs(out_ref[own1, :])),
        )
        r = [local_amax, local_amax]
        for h in range(max(AMAX_HOPS)):
            rdmas = []
            active = [d for d in (0, 1) if h < AMAX_HOPS[d]]
            for d in active:
                amax_sbuf[d, :, :] = jnp.full((1, 128), r[d], jnp.float32)
                rdma = pltpu.make_async_remote_copy(
                    src_ref=amax_sbuf.at[d],
                    dst_ref=amax_rbuf.at[d, h],
                    send_sem=ax_send_sems.at[d, h],
                    recv_sem=ax_recv_sems.at[d, h],
                    device_id=(to_nbr(d),),
                    device_id_type=pl.DeviceIdType.MESH,
                )
                rdma.start()
                rdmas.append(rdma)
            for d, rdma in zip(active, rdmas):
                rdma.wait()
                r[d] = jnp.maximum(r[d], amax_rbuf[d, h, 0, 0])
        amax = jnp.maximum(r[0], r[1])
        scale = amax / 448.0
        inv_scale = 448.0 / amax

        for rows in (own0, own1):
            q_ref[rows, :] = (out_ref[rows, :] * inv_scale).astype(
                jnp.float8_e4m3fn
            )

        def start_ag(t):
            rdmas = []
            for d in (0, 1):
                rc = rows_h(ag_send_chunk(d, t), d)
                rdma = pltpu.make_async_remote_copy(
                    src_ref=q_ref.at[rc, :],
                    dst_ref=q_ref.at[rc, :],
                    send_sem=ag_send_sems.at[d, t],
                    recv_sem=ag_recv_sems.at[d, t],
                    device_id=(to_nbr(d),),
                    device_id_type=pl.DeviceIdType.MESH,
                )
                rdma.start()
                rdmas.append(rdma)
            return rdmas

        rdmas = start_ag(0)
        for t in range(N_DEV - 1):
            for d in (0, 1):
                rdmas[d].wait()
            if t + 1 <= N_DEV - 2:
                rdmas = start_ag(t + 1)

        def deq_body(c, carry):
            q = q_ref[pl.ds(c * ch, ch), :]
            out_ref[pl.ds(c * ch, ch), :] = q.astype(jnp.float32) * scale
            return carry

        lax.fori_loop(0, N_DEV, deq_body, 0)

    return pl.pallas_call(
        body,
        out_shape=jax.ShapeDtypeStruct((m, n), jnp.float32),
        in_specs=[
            pl.BlockSpec(memory_space=pltpu.VMEM),
            pl.BlockSpec(memory_space=pltpu.VMEM),
        ],
        out_specs=pl.BlockSpec(memory_space=pltpu.VMEM),
        scratch_shapes=[
            pltpu.VMEM((2, half, n), jnp.bfloat16),
            pltpu.VMEM((2, half, n), jnp.bfloat16),
            pltpu.VMEM((m, n), jnp.float8_e4m3fn),
            pltpu.VMEM((2, 1, 128), jnp.float32),
            pltpu.VMEM((2, max(AMAX_HOPS), 1, 128), jnp.float32),
            pltpu.SemaphoreType.DMA((2, N_DEV - 1)),
            pltpu.SemaphoreType.DMA((2, N_DEV - 1)),
            pltpu.SemaphoreType.REGULAR((2,)),
            pltpu.SemaphoreType.DMA((2, max(AMAX_HOPS))),
            pltpu.SemaphoreType.DMA((2, max(AMAX_HOPS))),
            pltpu.SemaphoreType.DMA((2, N_DEV - 1)),
            pltpu.SemaphoreType.DMA((2, N_DEV - 1)),
        ],
        compiler_params=pltpu.CompilerParams(
            collective_id=0,
            vmem_limit_bytes=58 * 1024 * 1024,
        ),
    )(x, w_mat)


# device time: 191119 ns/iter; 3.7551x vs baseline; 1.1133x over previous
import jax
import jax.numpy as jnp
from jax import lax
from jax.experimental import pallas as pl
from jax.experimental.pallas import tpu as pltpu

N_DEV = 8
N_SUB = 2
AMAX_HOPS = (4, 3)


def kernel(x, w_mat):
    m, k_per = x.shape
    _, n = w_mat.shape
    ch = m // N_DEV
    half = ch // 2
    sub = n // N_SUB

    def body(x_ref, w_ref, out_ref, comm_ref, sbuf_ref, q_ref,
             amax_sbuf, amax_rbuf,
             rs_send_sems, rs_recv_sems, credit_sems,
             ax_send_sems, ax_recv_sems,
             ag_send_sems, ag_recv_sems):
        me = lax.axis_index("i")
        left = (me - 1) % N_DEV
        right = (me + 1) % N_DEV

        barrier_sem = pltpu.get_barrier_semaphore()
        for nbr in (left, right):
            pl.semaphore_signal(
                barrier_sem, inc=1,
                device_id=(nbr,), device_id_type=pl.DeviceIdType.MESH,
            )
        pl.semaphore_wait(barrier_sem, 2)

        def rows_h(c, d):
            return pl.ds(c * ch + d * half, half)

        def cols(u):
            return pl.ds(u * sub, sub)

        def rs_send_chunk(d, s):
            return (me - s) % N_DEV if d == 0 else (me + s) % N_DEV

        def rs_recv_chunk(d, s):
            return (me - s - 1) % N_DEV if d == 0 else (me + s + 1) % N_DEV

        def ag_send_chunk(d, t):
            return (me + 1 - t) % N_DEV if d == 0 else (me - 1 + t) % N_DEV

        def ag_recv_chunk(d, t):
            return (me - t) % N_DEV if d == 0 else (me + t) % N_DEV

        to_nbr = (lambda d: right if d == 0 else left)
        from_nbr = (lambda d: left if d == 0 else right)

        def fill_sbuf(d, u, s):
            sbuf_ref[d, u] = out_ref[
                rows_h(rs_send_chunk(d, s), d), cols(u)
            ].astype(jnp.bfloat16)

        def start_rs(d, u, s):
            rdma = pltpu.make_async_remote_copy(
                src_ref=sbuf_ref.at[d, u],
                dst_ref=comm_ref.at[d, u],
                send_sem=rs_send_sems.at[d, s, u],
                recv_sem=rs_recv_sems.at[d, s, u],
                device_id=(to_nbr(d),),
                device_id_type=pl.DeviceIdType.MESH,
            )
            rdma.start()
            return rdma

        out_ref[pl.ds(me * ch, ch), :] = jnp.dot(
            x_ref[pl.ds(me * ch, ch), :], w_ref[...],
            preferred_element_type=jnp.float32,
        )
        rdmas = {}
        for u in (0, 1):
            for d in (0, 1):
                fill_sbuf(d, u, 0)
                rdmas[d, u] = start_rs(d, u, 0)

        def gemm_body(j, carry):
            c = (me - j) % N_DEV
            out_ref[pl.ds(c * ch, ch), :] = jnp.dot(
                x_ref[pl.ds(c * ch, ch), :], w_ref[...],
                preferred_element_type=jnp.float32,
            )
            return carry

        lax.fori_loop(1, N_DEV, gemm_body, 0)

        for s in range(N_DEV - 1):
            for u in (0, 1):
                for d in (0, 1):
                    rdmas[d, u].wait()
                    out_ref[rows_h(rs_recv_chunk(d, s), d), cols(u)] += (
                        comm_ref[d, u].astype(jnp.float32)
                    )
                    if s <= N_DEV - 3:
                        pl.semaphore_signal(
                            credit_sems.at[d, u], inc=1,
                            device_id=(from_nbr(d),),
                            device_id_type=pl.DeviceIdType.MESH,
                        )
                    if s + 1 <= N_DEV - 2:
                        fill_sbuf(d, u, s + 1)
                        pl.semaphore_wait(credit_sems.at[d, u], 1)
                        rdmas[d, u] = start_rs(d, u, s + 1)

        own0 = rows_h((me + 1) % N_DEV, 0)
        own1 = rows_h((me - 1) % N_DEV, 1)
        local_amax = jnp.maximum(
            jnp.max(jnp.abs(out_ref[own0, :])),
            jnp.max(jnp.abs(out_ref[own1, :])),
        )
        r = [local_amax, local_amax]
        for h in range(max(AMAX_HOPS)):
            hop_rdmas = []
            active = [d for d in (0, 1) if h < AMAX_HOPS[d]]
            for d in active:
                amax_sbuf[d, :, :] = jnp.full((1, 128), r[d], jnp.float32)
                rdma = pltpu.make_async_remote_copy(
                    src_ref=amax_sbuf.at[d],
                    dst_ref=amax_rbuf.at[d, h],
                    send_sem=ax_send_sems.at[d, h],
                    recv_sem=ax_recv_sems.at[d, h],
                    device_id=(to_nbr(d),),
                    device_id_type=pl.DeviceIdType.MESH,
                )
                rdma.start()
                hop_rdmas.append(rdma)
            for d, rdma in zip(active, hop_rdmas):
                rdma.wait()
                r[d] = jnp.maximum(r[d], amax_rbuf[d, h, 0, 0])
        amax = jnp.maximum(r[0], r[1])
        scale = amax / 448.0
        inv_scale = 448.0 / amax

        for rows in (own0, own1):
            y = out_ref[rows, :]
            q = (y * inv_scale).astype(jnp.float8_e4m3fn)
            q_ref[rows, :] = q
            out_ref[rows, :] = q.astype(jnp.float32) * scale

        def start_ag(t):
            ag = []
            for d in (0, 1):
                rc = rows_h(ag_send_chunk(d, t), d)
                rdma = pltpu.make_async_remote_copy(
                    src_ref=q_ref.at[rc, :],
                    dst_ref=q_ref.at[rc, :],
                    send_sem=ag_send_sems.at[d, t],
                    recv_sem=ag_recv_sems.at[d, t],
                    device_id=(to_nbr(d),),
                    device_id_type=pl.DeviceIdType.MESH,
                )
                rdma.start()
                ag.append(rdma)
            return ag

        ag_rdmas = start_ag(0)
        for t in range(N_DEV - 1):
            for d in (0, 1):
                ag_rdmas[d].wait()
            if t + 1 <= N_DEV - 2:
                ag_rdmas = start_ag(t + 1)
            for d in (0, 1):
                rc = rows_h(ag_recv_chunk(d, t), d)
                out_ref[rc, :] = q_ref[rc, :].astype(jnp.float32) * scale

    return pl.pallas_call(
        body,
        out_shape=jax.ShapeDtypeStruct((m, n), jnp.float32),
        in_specs=[
            pl.BlockSpec(memory_space=pltpu.VMEM),
            pl.BlockSpec(memory_space=pltpu.VMEM),
        ],
        out_specs=pl.BlockSpec(memory_space=pltpu.VMEM),
        scratch_shapes=[
            pltpu.VMEM((2, N_SUB, half, sub), jnp.bfloat16),
            pltpu.VMEM((2, N_SUB, half, sub), jnp.bfloat16),
            pltpu.VMEM((m, n), jnp.float8_e4m3fn),
            pltpu.VMEM((2, 1, 128), jnp.float32),
            pltpu.VMEM((2, max(AMAX_HOPS), 1, 128), jnp.float32),
            pltpu.SemaphoreType.DMA((2, N_DEV - 1, N_SUB)),
            pltpu.SemaphoreType.DMA((2, N_DEV - 1, N_SUB)),
            pltpu.SemaphoreType.REGULAR((2, N_SUB)),
            pltpu.SemaphoreType.DMA((2, max(AMAX_HOPS))),
            pltpu.SemaphoreType.DMA((2, max(AMAX_HOPS))),
            pltpu.SemaphoreType.DMA((2, N_DEV - 1)),
            pltpu.SemaphoreType.DMA((2, N_DEV - 1)),
        ],
        compiler_params=pltpu.CompilerParams(
            collective_id=0,
            vmem_limit_bytes=60 * 1024 * 1024,
        ),
    )(x, w_mat)


# device time: 178578 ns/iter; 4.0188x vs baseline; 1.0702x over previous
import jax
import jax.numpy as jnp
from jax import lax
from jax.experimental import pallas as pl
from jax.experimental.pallas import tpu as pltpu

N_DEV = 8
N_SUB = 2
AMAX_HOPS = (4, 3)


def kernel(x, w_mat):
    m, k_per = x.shape
    _, n = w_mat.shape
    ch = m // N_DEV
    half = ch // 2
    sub = n // N_SUB

    def body(x_ref, w_ref, out_ref, comm_ref, sbuf_ref, q_ref,
             amax_sbuf, amax_rbuf,
             rs_send_sems, rs_recv_sems, credit_sems,
             ax_send_sems, ax_recv_sems,
             ag_send_sems, ag_recv_sems):
        me = lax.axis_index("i")
        left = (me - 1) % N_DEV
        right = (me + 1) % N_DEV

        barrier_sem = pltpu.get_barrier_semaphore()
        for nbr in (left, right):
            pl.semaphore_signal(
                barrier_sem, inc=1,
                device_id=(nbr,), device_id_type=pl.DeviceIdType.MESH,
            )
        pl.semaphore_wait(barrier_sem, 2)

        def rows_h(c, d):
            return pl.ds(c * ch + d * half, half)

        def cols(u):
            return pl.ds(u * sub, sub)

        def rs_send_chunk(d, s):
            return (me - s) % N_DEV if d == 0 else (me + s) % N_DEV

        def rs_recv_chunk(d, s):
            return (me - s - 1) % N_DEV if d == 0 else (me + s + 1) % N_DEV

        def ag_send_chunk(d, t):
            return (me + 1 - t) % N_DEV if d == 0 else (me - 1 + t) % N_DEV

        def ag_recv_chunk(d, t):
            return (me - t) % N_DEV if d == 0 else (me + t) % N_DEV

        to_nbr = (lambda d: right if d == 0 else left)
        from_nbr = (lambda d: left if d == 0 else right)

        def fill_sbuf(d, u, s):
            sbuf_ref[d, u] = out_ref[
                rows_h(rs_send_chunk(d, s), d), cols(u)
            ].astype(jnp.bfloat16)

        def start_rs(d, u, s):
            rdma = pltpu.make_async_remote_copy(
                src_ref=sbuf_ref.at[d, u],
                dst_ref=comm_ref.at[d, u],
                send_sem=rs_send_sems.at[d, s, u],
                recv_sem=rs_recv_sems.at[d, s, u],
                device_id=(to_nbr(d),),
                device_id_type=pl.DeviceIdType.MESH,
            )
            rdma.start()
            return rdma

        out_ref[pl.ds(me * ch, ch), :] = jnp.dot(
            x_ref[pl.ds(me * ch, ch), :], w_ref[...],
            preferred_element_type=jnp.float32,
        )
        rdmas = {}
        for u in (0, 1):
            for d in (0, 1):
                fill_sbuf(d, u, 0)
                rdmas[d, u] = start_rs(d, u, 0)

        def gemm_body(j, carry):
            c = (me - j) % N_DEV
            out_ref[pl.ds(c * ch, ch), :] = jnp.dot(
                x_ref[pl.ds(c * ch, ch), :], w_ref[...],
                preferred_element_type=jnp.float32,
            )
            return carry

        lax.fori_loop(1, N_DEV, gemm_body, 0)

        for s in range(N_DEV - 1):
            for u in (0, 1):
                for d in (0, 1):
                    rdmas[d, u].wait()
                    out_ref[rows_h(rs_recv_chunk(d, s), d), cols(u)] += (
                        comm_ref[d, u].astype(jnp.float32)
                    )
                    if s <= N_DEV - 3:
                        pl.semaphore_signal(
                            credit_sems.at[d, u], inc=1,
                            device_id=(from_nbr(d),),
                            device_id_type=pl.DeviceIdType.MESH,
                        )
                    if s + 1 <= N_DEV - 2:
                        fill_sbuf(d, u, s + 1)
                        pl.semaphore_wait(credit_sems.at[d, u], 1)
                        rdmas[d, u] = start_rs(d, u, s + 1)

        own0 = rows_h((me + 1) % N_DEV, 0)
        own1 = rows_h((me - 1) % N_DEV, 1)
        local_amax = jnp.maximum(
            jnp.max(jnp.abs(out_ref[own0, :])),
            jnp.max(jnp.abs(out_ref[own1, :])),
        )
        r = [local_amax, local_amax]
        for h in range(max(AMAX_HOPS)):
            hop_rdmas = []
            active = [d for d in (0, 1) if h < AMAX_HOPS[d]]
            for d in active:
                amax_sbuf[d, :, :] = jnp.full((1, 128), r[d], jnp.float32)
                rdma = pltpu.make_async_remote_copy(
                    src_ref=amax_sbuf.at[d],
                    dst_ref=amax_rbuf.at[d, h],
                    send_sem=ax_send_sems.at[d, h],
                    recv_sem=ax_recv_sems.at[d, h],
                    device_id=(to_nbr(d),),
                    device_id_type=pl.DeviceIdType.MESH,
                )
                rdma.start()
                hop_rdmas.append(rdma)
            for d, rdma in zip(active, hop_rdmas):
                rdma.wait()
                r[d] = jnp.maximum(r[d], amax_rbuf[d, h, 0, 0])
        amax = jnp.maximum(r[0], r[1])
        scale = amax / 448.0
        inv_scale = 448.0 / amax

        for rows in (own0, own1):
            y = out_ref[rows, :]
            q = (y * inv_scale).astype(jnp.float8_e4m3fn)
            q_ref[rows, :] = q
            out_ref[rows, :] = q.astype(jnp.float32) * scale

        def start_ag(d, t, u):
            rc = rows_h(ag_send_chunk(d, t), d)
            rdma = pltpu.make_async_remote_copy(
                src_ref=q_ref.at[rc, cols(u)],
                dst_ref=q_ref.at[rc, cols(u)],
                send_sem=ag_send_sems.at[d, t, u],
                recv_sem=ag_recv_sems.at[d, t, u],
                device_id=(to_nbr(d),),
                device_id_type=pl.DeviceIdType.MESH,
            )
            rdma.start()
            return rdma

        ag_rdmas = {}
        for u in (0, 1):
            for d in (0, 1):
                ag_rdmas[d, u] = start_ag(d, 0, u)
        for t in range(N_DEV - 1):
            for u in (0, 1):
                for d in (0, 1):
                    ag_rdmas[d, u].wait()
                    if t + 1 <= N_DEV - 2:
                        ag_rdmas[d, u] = start_ag(d, t + 1, u)
            for d in (0, 1):
                rc = rows_h(ag_recv_chunk(d, t), d)
                out_ref[rc, :] = q_ref[rc, :].astype(jnp.float32) * scale

    return pl.pallas_call(
        body,
        out_shape=jax.ShapeDtypeStruct((m, n), jnp.float32),
        in_specs=[
            pl.BlockSpec(memory_space=pltpu.VMEM),
            pl.BlockSpec(memory_space=pltpu.VMEM),
        ],
        out_specs=pl.BlockSpec(memory_space=pltpu.VMEM),
        scratch_shapes=[
            pltpu.VMEM((2, N_SUB, half, sub), jnp.bfloat16),
            pltpu.VMEM((2, N_SUB, half, sub), jnp.bfloat16),
            pltpu.VMEM((m, n), jnp.float8_e4m3fn),
            pltpu.VMEM((2, 1, 128), jnp.float32),
            pltpu.VMEM((2, max(AMAX_HOPS), 1, 128), jnp.float32),
            pltpu.SemaphoreType.DMA((2, N_DEV - 1, N_SUB)),
            pltpu.SemaphoreType.DMA((2, N_DEV - 1, N_SUB)),
            pltpu.SemaphoreType.REGULAR((2, N_SUB)),
            pltpu.SemaphoreType.DMA((2, max(AMAX_HOPS))),
            pltpu.SemaphoreType.DMA((2, max(AMAX_HOPS))),
            pltpu.SemaphoreType.DMA((2, N_DEV - 1, N_SUB)),
            pltpu.SemaphoreType.DMA((2, N_DEV - 1, N_SUB)),
        ],
        compiler_params=pltpu.CompilerParams(
            collective_id=0,
            vmem_limit_bytes=60 * 1024 * 1024,
        ),
    )(x, w_mat)
